# baseline (speedup 1.0000x reference)
import jax
import jax.numpy as jnp
from jax import lax
from jax.experimental import pallas as pl
from jax.experimental.pallas import tpu as pltpu
from jax.experimental.pallas import tpu_sc as plsc

_NB = 4
_NBUF = 4
_NCHUNK = 4


def _sc_gather_chunk(idx1d, weight, b_start, batch_c, fields, embed_dim):
    mesh = plsc.VectorSubcoreMesh(
        core_axis_name="core", subcore_axis_name="subcore"
    )
    info = plsc.get_sparse_core_info()
    nw = info.num_cores * info.num_subcores
    window = _NB * fields
    b_per_w = batch_c // nw
    steps = b_per_w // _NB
    groups = steps // _NBUF - 1
    idx_per_w = b_per_w * fields

    @pl.kernel(
        out_type=jax.ShapeDtypeStruct(
            (batch_c * fields, embed_dim), weight.dtype
        ),
        mesh=mesh,
        scratch_types=[
            pltpu.VMEM((idx_per_w,), jnp.int32),
            pltpu.VMEM((_NBUF, window, embed_dim), jnp.float32),
            pltpu.SemaphoreType.DMA((_NBUF,)),
            pltpu.SemaphoreType.DMA((_NBUF,)),
        ],
    )
    def gather_kernel(x_hbm, i_hbm, o_hbm, idx_v, rows_v, gsem, wsem):
        c = lax.axis_index("core")
        s = lax.axis_index("subcore")
        wid = s * info.num_cores + c
        pltpu.sync_copy(
            i_hbm.at[
                pl.ds(b_start * fields + wid * idx_per_w, idx_per_w)
            ],
            idx_v,
        )
        r_base = wid * idx_per_w

        def issue_gather(step, nb):
            off = pl.multiple_of(step * window, 8)
            pltpu.async_copy(
                x_hbm.at[idx_v.at[pl.ds(off, window)]],
                rows_v.at[nb],
                gsem.at[nb],
            )

        def wait_gather(nb):
            pltpu.make_async_copy(
                x_hbm.at[idx_v.at[pl.ds(0, window)]],
                rows_v.at[nb],
                gsem.at[nb],
            ).wait()

        def issue_write(step, nb):
            off = pl.multiple_of(r_base + step * window, 8)
            pltpu.async_copy(
                rows_v.at[nb],
                o_hbm.at[pl.ds(off, window)],
                wsem.at[nb],
            )

        def wait_write(nb):
            pltpu.make_async_copy(
                rows_v.at[nb],
                o_hbm.at[pl.ds(0, window)],
                wsem.at[nb],
            ).wait()

        for nb in range(_NBUF):
            issue_gather(nb, nb)

        @pl.loop(0, groups)
        def _(grp):
            base = grp * _NBUF
            for nb in range(_NBUF):
                wait_gather(nb)
                issue_write(base + nb, nb)
            for nb in range(_NBUF):
                wait_write(nb)
                issue_gather(base + _NBUF + nb, nb)

        base = groups * _NBUF
        for nb in range(_NBUF):
            wait_gather(nb)
            issue_write(base + nb, nb)
        for nb in range(_NBUF):
            wait_write(nb)

    return gather_kernel(weight, idx1d)


def kernel(indices, weight):
    batch, fields = indices.shape
    vocab, embed_dim = weight.shape
    idx1d = indices.reshape(batch * fields).astype(jnp.int32)
    batch_c = batch // _NCHUNK
    acc = jnp.zeros((batch, fields, embed_dim), weight.dtype)
    for c in range(_NCHUNK):
        chunk = _sc_gather_chunk(
            idx1d, weight, c * batch_c, batch_c, fields, embed_dim
        ).reshape(batch_c, fields, embed_dim)
        acc = lax.dynamic_update_slice(acc, chunk, (c * batch_c, 0, 0))
    return acc


# speedup vs baseline: 1.3634x; 1.3634x over previous
import jax
import jax.numpy as jnp
from jax import lax
from jax.experimental import pallas as pl
from jax.experimental.pallas import tpu as pltpu
from jax.experimental.pallas import tpu_sc as plsc

_NB = 4
_NBUF = 4
_NCHUNK = 4


def _sc_gather_chunk(idx1d, weight, b_start, batch_c, fields, embed_dim):
    mesh = plsc.VectorSubcoreMesh(
        core_axis_name="core", subcore_axis_name="subcore"
    )
    info = plsc.get_sparse_core_info()
    nw = info.num_cores * info.num_subcores
    window = _NB * fields
    b_per_w = batch_c // nw
    steps = b_per_w // _NB
    groups = steps // _NBUF - 1
    idx_per_w = b_per_w * fields

    @pl.kernel(
        out_type=jax.ShapeDtypeStruct(
            (batch_c * fields, embed_dim), weight.dtype
        ),
        mesh=mesh,
        scratch_types=[
            pltpu.VMEM((idx_per_w,), jnp.int32),
            pltpu.VMEM((_NBUF, window, embed_dim), jnp.float32),
            pltpu.SemaphoreType.DMA((_NBUF,)),
            pltpu.SemaphoreType.DMA((_NBUF,)),
        ],
    )
    def gather_kernel(x_hbm, i_hbm, o_hbm, idx_v, rows_v, gsem, wsem):
        c = lax.axis_index("core")
        s = lax.axis_index("subcore")
        wid = s * info.num_cores + c
        pltpu.sync_copy(
            i_hbm.at[
                pl.ds(b_start * fields + wid * idx_per_w, idx_per_w)
            ],
            idx_v,
        )
        r_base = wid * idx_per_w

        def issue_gather(step, nb):
            off = pl.multiple_of(step * window, 8)
            pltpu.async_copy(
                x_hbm.at[idx_v.at[pl.ds(off, window)]],
                rows_v.at[nb],
                gsem.at[nb],
            )

        def wait_gather(nb):
            pltpu.make_async_copy(
                x_hbm.at[idx_v.at[pl.ds(0, window)]],
                rows_v.at[nb],
                gsem.at[nb],
            ).wait()

        def issue_write(step, nb):
            off = pl.multiple_of(r_base + step * window, 8)
            pltpu.async_copy(
                rows_v.at[nb],
                o_hbm.at[pl.ds(off, window)],
                wsem.at[nb],
            )

        def wait_write(nb):
            pltpu.make_async_copy(
                rows_v.at[nb],
                o_hbm.at[pl.ds(0, window)],
                wsem.at[nb],
            ).wait()

        for nb in range(_NBUF):
            issue_gather(nb, nb)

        @pl.loop(0, groups)
        def _(grp):
            base = grp * _NBUF
            for nb in range(_NBUF):
                wait_gather(nb)
                issue_write(base + nb, nb)
            for nb in range(_NBUF):
                wait_write(nb)
                issue_gather(base + _NBUF + nb, nb)

        base = groups * _NBUF
        for nb in range(_NBUF):
            wait_gather(nb)
            issue_write(base + nb, nb)
        for nb in range(_NBUF):
            wait_write(nb)

    return gather_kernel(weight, idx1d)


def kernel(indices, weight):
    batch, fields = indices.shape
    vocab, embed_dim = weight.shape
    idx1d = indices.reshape(batch * fields).astype(jnp.int32)
    out2d = _sc_gather_chunk(idx1d, weight, 0, batch, fields, embed_dim)
    return out2d.reshape(batch, fields, embed_dim)


# speedup vs baseline: 2.2739x; 1.6679x over previous
import jax
import jax.numpy as jnp
from jax import lax
from jax.experimental import pallas as pl
from jax.experimental.pallas import tpu as pltpu
from jax.experimental.pallas import tpu_sc as plsc

_NB = 4
_NBUF = 4
_NCHUNK = 4


def _sc_gather_chunk(idx1d, weight, b_start, batch_c, fields, embed_dim):
    mesh = plsc.VectorSubcoreMesh(
        core_axis_name="core", subcore_axis_name="subcore"
    )
    info = plsc.get_sparse_core_info()
    nw = info.num_cores * info.num_subcores
    window = _NB * fields
    b_per_w = batch_c // nw
    steps = b_per_w // _NB
    groups = steps // _NBUF - 1
    idx_per_w = b_per_w * fields

    @pl.kernel(
        out_type=jax.ShapeDtypeStruct(
            (batch_c, fields, embed_dim), weight.dtype
        ),
        mesh=mesh,
        scratch_types=[
            pltpu.VMEM((idx_per_w,), jnp.int32),
            pltpu.VMEM((_NBUF, window, embed_dim), jnp.float32),
            pltpu.SemaphoreType.DMA((_NBUF,)),
            pltpu.SemaphoreType.DMA((_NBUF,)),
        ],
    )
    def gather_kernel(x_hbm, i_hbm, o_hbm, idx_v, rows_v, gsem, wsem):
        c = lax.axis_index("core")
        s = lax.axis_index("subcore")
        wid = s * info.num_cores + c
        pltpu.sync_copy(
            i_hbm.at[
                pl.ds(b_start * fields + wid * idx_per_w, idx_per_w)
            ],
            idx_v,
        )

        def issue_gather(step, nb):
            off = pl.multiple_of(step * window, 8)
            pltpu.async_copy(
                x_hbm.at[idx_v.at[pl.ds(off, window)]],
                rows_v.at[nb],
                gsem.at[nb],
            )

        def wait_gather(nb):
            pltpu.make_async_copy(
                x_hbm.at[idx_v.at[pl.ds(0, window)]],
                rows_v.at[nb],
                gsem.at[nb],
            ).wait()

        b_base = wid * b_per_w

        def issue_write(step, nb):
            pltpu.async_copy(
                rows_v.at[nb].reshape(_NB, fields, embed_dim),
                o_hbm.at[pl.ds(b_base + step * _NB, _NB)],
                wsem.at[nb],
            )

        def wait_write(nb):
            pltpu.make_async_copy(
                rows_v.at[nb].reshape(_NB, fields, embed_dim),
                o_hbm.at[pl.ds(b_base, _NB)],
                wsem.at[nb],
            ).wait()

        for nb in range(_NBUF):
            issue_gather(nb, nb)

        @pl.loop(0, groups)
        def _(grp):
            base = grp * _NBUF
            for nb in range(_NBUF):
                wait_gather(nb)
                issue_write(base + nb, nb)
            for nb in range(_NBUF):
                wait_write(nb)
                issue_gather(base + _NBUF + nb, nb)

        base = groups * _NBUF
        for nb in range(_NBUF):
            wait_gather(nb)
            issue_write(base + nb, nb)
        for nb in range(_NBUF):
            wait_write(nb)

    return gather_kernel(weight, idx1d)


def kernel(indices, weight):
    batch, fields = indices.shape
    vocab, embed_dim = weight.shape
    idx1d = indices.reshape(batch * fields).astype(jnp.int32)
    return _sc_gather_chunk(idx1d, weight, 0, batch, fields, embed_dim)


# speedup vs baseline: 2.2769x; 1.0013x over previous
import jax
import jax.numpy as jnp
from jax import lax
from jax.experimental import pallas as pl
from jax.experimental.pallas import tpu as pltpu
from jax.experimental.pallas import tpu_sc as plsc

_NB = 4
_NBUF = 8
_NCHUNK = 4


def _sc_gather_chunk(idx1d, weight, b_start, batch_c, fields, embed_dim):
    mesh = plsc.VectorSubcoreMesh(
        core_axis_name="core", subcore_axis_name="subcore"
    )
    info = plsc.get_sparse_core_info()
    nw = info.num_cores * info.num_subcores
    window = _NB * fields
    b_per_w = batch_c // nw
    steps = b_per_w // _NB
    groups = steps // _NBUF - 1
    idx_per_w = b_per_w * fields

    @pl.kernel(
        out_type=jax.ShapeDtypeStruct(
            (batch_c, fields, embed_dim), weight.dtype
        ),
        mesh=mesh,
        scratch_types=[
            pltpu.VMEM((idx_per_w,), jnp.int32),
            pltpu.VMEM((_NBUF, window, embed_dim), jnp.float32),
            pltpu.SemaphoreType.DMA((_NBUF,)),
            pltpu.SemaphoreType.DMA((_NBUF,)),
        ],
    )
    def gather_kernel(x_hbm, i_hbm, o_hbm, idx_v, rows_v, gsem, wsem):
        c = lax.axis_index("core")
        s = lax.axis_index("subcore")
        wid = s * info.num_cores + c
        pltpu.sync_copy(
            i_hbm.at[
                pl.ds(b_start * fields + wid * idx_per_w, idx_per_w)
            ],
            idx_v,
        )

        def issue_gather(step, nb):
            off = pl.multiple_of(step * window, 8)
            pltpu.async_copy(
                x_hbm.at[idx_v.at[pl.ds(off, window)]],
                rows_v.at[nb],
                gsem.at[nb],
            )

        def wait_gather(nb):
            pltpu.make_async_copy(
                x_hbm.at[idx_v.at[pl.ds(0, window)]],
                rows_v.at[nb],
                gsem.at[nb],
            ).wait()

        b_base = wid * b_per_w

        def issue_write(step, nb):
            pltpu.async_copy(
                rows_v.at[nb].reshape(_NB, fields, embed_dim),
                o_hbm.at[pl.ds(b_base + step * _NB, _NB)],
                wsem.at[nb],
            )

        def wait_write(nb):
            pltpu.make_async_copy(
                rows_v.at[nb].reshape(_NB, fields, embed_dim),
                o_hbm.at[pl.ds(b_base, _NB)],
                wsem.at[nb],
            ).wait()

        for nb in range(_NBUF):
            issue_gather(nb, nb)

        @pl.loop(0, groups)
        def _(grp):
            base = grp * _NBUF
            for nb in range(_NBUF):
                wait_gather(nb)
                issue_write(base + nb, nb)
            for nb in range(_NBUF):
                wait_write(nb)
                issue_gather(base + _NBUF + nb, nb)

        base = groups * _NBUF
        for nb in range(_NBUF):
            wait_gather(nb)
            issue_write(base + nb, nb)
        for nb in range(_NBUF):
            wait_write(nb)

    return gather_kernel(weight, idx1d)


def kernel(indices, weight):
    batch, fields = indices.shape
    vocab, embed_dim = weight.shape
    idx1d = indices.reshape(batch * fields).astype(jnp.int32)
    return _sc_gather_chunk(idx1d, weight, 0, batch, fields, embed_dim)
